# baseline (speedup 1.0000x reference)
import functools

import jax
import jax.numpy as jnp
from jax import lax
from jax.experimental import pallas as pl
from jax.experimental.pallas import tpu as pltpu
from jax.experimental.pallas import tpu_sc as plsc

_B, _D = 16384, 2048
_H = 2048


_BR = 512
_G = _B // _BR


def _mse_kernel(yh_ref, y_ref, out_ref, acc_ref):
    i = pl.program_id(0)

    @pl.when(i == 0)
    def _init():
        acc_ref[0] = 0.0

    d = yh_ref[...] - y_ref[...]
    acc_ref[0] += jnp.sum(d * d)

    @pl.when(i == _G - 1)
    def _fin():
        out_ref[0] = acc_ref[0]


def _tc_sumsq_diff(y_hat, y):
    return pl.pallas_call(
        _mse_kernel,
        grid=(_G,),
        in_specs=[
            pl.BlockSpec((_BR, _D), lambda i: (i, 0)),
            pl.BlockSpec((_BR, _D), lambda i: (i, 0)),
        ],
        out_specs=pl.BlockSpec(memory_space=pltpu.SMEM),
        out_shape=jax.ShapeDtypeStruct((1,), jnp.float32),
        scratch_shapes=[pltpu.SMEM((1,), jnp.float32)],
        compiler_params=pltpu.CompilerParams(
            dimension_semantics=("arbitrary",),
        ),
    )(y_hat, y)


_NC, _NS = 2, 16
_NW = _NC * _NS
_RPW = _H // _NW
_CR = 8
_NCHUNK = _RPW // _CR


def _sc_body(w1_hbm, w2_hbm, out_hbm, buf_a, buf_b, acc_v, sem_a, sem_b):
    wid = lax.axis_index("s") * _NC + lax.axis_index("c")
    row0 = wid * _RPW

    for a_idx, w_hbm in ((0, w1_hbm), (1, w2_hbm)):
        accs = [jnp.zeros((16,), jnp.float32) for _ in range(_CR)]
        cps = [pltpu.async_copy(
            w_hbm.at[pl.ds(row0, _CR), :], buf_a, sem_a)]
        for c in range(_NCHUNK):
            if c + 1 < _NCHUNK:
                nb, ns = (buf_b, sem_b) if (c % 2 == 0) else (buf_a, sem_a)
                cps.append(pltpu.async_copy(
                    w_hbm.at[pl.ds(row0 + (c + 1) * _CR, _CR), :], nb, ns))
            cps[c].wait()
            buf = buf_a if (c % 2 == 0) else buf_b

            def body(j, a, _buf=buf):
                new = []
                for r in range(_CR):
                    v = _buf[r, pl.ds(j * 16, 16)]
                    new.append(a[r] + v * v)
                return tuple(new)
            accs = lax.fori_loop(0, _D // 16, body, tuple(accs))

        total = accs[0]
        for r in range(1, _CR):
            total = total + accs[r]
        acc_v[...] = total
        pltpu.sync_copy(acc_v, out_hbm.at[a_idx, wid])


def _sc_w_sumsq(w1, w2):
    mesh = plsc.VectorSubcoreMesh(
        core_axis_name="c", subcore_axis_name="s",
        num_cores=_NC, num_subcores=_NS)
    kfn = pl.kernel(
        _sc_body,
        out_type=jax.ShapeDtypeStruct((2, _NW, 16), jnp.float32),
        mesh=mesh,
        scratch_types=[
            pltpu.VMEM((_CR, _D), jnp.float32),
            pltpu.VMEM((_CR, _D), jnp.float32),
            pltpu.VMEM((16,), jnp.float32),
            pltpu.SemaphoreType.DMA,
            pltpu.SemaphoreType.DMA,
        ],
        cost_estimate=pl.CostEstimate(
            flops=2 * 2 * _H * _D,
            bytes_accessed=2 * 4 * _H * _D,
            transcendentals=0,
        ),
    )
    return kfn(w1, w2)


def kernel(y_hat, y, W1, W2):
    w_part = _sc_w_sumsq(W1, W2)
    sumsq = _tc_sumsq_diff(y_hat, y)
    mse = sumsq[0] / (_B * _D)
    pen = jnp.sqrt(jnp.sum(w_part[0])) + jnp.sqrt(jnp.sum(w_part[1]))
    return mse + pen


# speedup vs baseline: 1.2314x; 1.2314x over previous
import jax
import jax.numpy as jnp
from jax.experimental import pallas as pl
from jax.experimental.pallas import tpu as pltpu

_B, _D = 16384, 2048
_H = 2048
_BR = 512
_G = _B // _BR
_WR = _H // _G


def _loss_kernel(yh_ref, y_ref, w1_ref, w2_ref, out_ref, acc_ref):
    i = pl.program_id(0)

    @pl.when(i == 0)
    def _init():
        acc_ref[0] = 0.0
        acc_ref[1] = 0.0
        acc_ref[2] = 0.0

    d = yh_ref[...] - y_ref[...]
    acc_ref[0] += jnp.sum(d * d)
    w1 = w1_ref[...]
    acc_ref[1] += jnp.sum(w1 * w1)
    w2 = w2_ref[...]
    acc_ref[2] += jnp.sum(w2 * w2)

    @pl.when(i == _G - 1)
    def _fin():
        out_ref[0, 0] = (acc_ref[0] / (_B * _D)
                         + jnp.sqrt(acc_ref[1]) + jnp.sqrt(acc_ref[2]))


def kernel(y_hat, y, W1, W2):
    out = pl.pallas_call(
        _loss_kernel,
        grid=(_G,),
        in_specs=[
            pl.BlockSpec((_BR, _D), lambda i: (i, 0)),
            pl.BlockSpec((_BR, _D), lambda i: (i, 0)),
            pl.BlockSpec((_WR, _D), lambda i: (i, 0)),
            pl.BlockSpec((_WR, _H), lambda i: (i, 0)),
        ],
        out_specs=pl.BlockSpec(memory_space=pltpu.SMEM),
        out_shape=jax.ShapeDtypeStruct((1, 1), jnp.float32),
        scratch_shapes=[pltpu.SMEM((3,), jnp.float32)],
        compiler_params=pltpu.CompilerParams(
            dimension_semantics=("arbitrary",),
        ),
    )(y_hat, y, W1, W2)
    return out[0, 0]


# speedup vs baseline: 1.2433x; 1.0096x over previous
import jax
import jax.numpy as jnp
from jax import lax
from jax.experimental import pallas as pl
from jax.experimental.pallas import tpu as pltpu

_B, _D = 16384, 2048
_H = 2048
_BR = 512
_G = _B // _BR
_WR = _H // _G
_NBUF = 3


def _loss_kernel(yh_hbm, y_hbm, w1_hbm, w2_hbm, out_ref,
                 yh_b, y_b, w1_b, w2_b, acc_ref,
                 s_yh, s_y, s_w1, s_w2):

    def fetch(s, b):
        pltpu.make_async_copy(
            yh_hbm.at[pl.ds(s * _BR, _BR)], yh_b.at[b], s_yh.at[b]).start()
        pltpu.make_async_copy(
            y_hbm.at[pl.ds(s * _BR, _BR)], y_b.at[b], s_y.at[b]).start()
        pltpu.make_async_copy(
            w1_hbm.at[pl.ds(s * _WR, _WR)], w1_b.at[b], s_w1.at[b]).start()
        pltpu.make_async_copy(
            w2_hbm.at[pl.ds(s * _WR, _WR)], w2_b.at[b], s_w2.at[b]).start()

    for s in range(_NBUF):
        fetch(s, s)

    acc_ref[0] = 0.0
    acc_ref[1] = 0.0
    acc_ref[2] = 0.0

    def step(s, carry):
        b = lax.rem(s, _NBUF)
        pltpu.make_async_copy(
            yh_hbm.at[pl.ds(0, _BR)], yh_b.at[b], s_yh.at[b]).wait()
        pltpu.make_async_copy(
            y_hbm.at[pl.ds(0, _BR)], y_b.at[b], s_y.at[b]).wait()
        pltpu.make_async_copy(
            w1_hbm.at[pl.ds(0, _WR)], w1_b.at[b], s_w1.at[b]).wait()
        pltpu.make_async_copy(
            w2_hbm.at[pl.ds(0, _WR)], w2_b.at[b], s_w2.at[b]).wait()

        d = yh_b[b] - y_b[b]
        acc_ref[0] += jnp.sum(d * d)
        w1 = w1_b[b]
        acc_ref[1] += jnp.sum(w1 * w1)
        w2 = w2_b[b]
        acc_ref[2] += jnp.sum(w2 * w2)

        @pl.when(s + _NBUF < _G)
        def _():
            nxt = s + _NBUF
            pltpu.make_async_copy(
                yh_hbm.at[pl.ds(nxt * _BR, _BR)], yh_b.at[b], s_yh.at[b]).start()
            pltpu.make_async_copy(
                y_hbm.at[pl.ds(nxt * _BR, _BR)], y_b.at[b], s_y.at[b]).start()
            pltpu.make_async_copy(
                w1_hbm.at[pl.ds(nxt * _WR, _WR)], w1_b.at[b], s_w1.at[b]).start()
            pltpu.make_async_copy(
                w2_hbm.at[pl.ds(nxt * _WR, _WR)], w2_b.at[b], s_w2.at[b]).start()
        return carry

    lax.fori_loop(0, _G, step, 0)

    out_ref[0, 0] = (acc_ref[0] / (_B * _D)
                     + jnp.sqrt(acc_ref[1]) + jnp.sqrt(acc_ref[2]))


def kernel(y_hat, y, W1, W2):
    out = pl.pallas_call(
        _loss_kernel,
        in_specs=[
            pl.BlockSpec(memory_space=pl.ANY),
            pl.BlockSpec(memory_space=pl.ANY),
            pl.BlockSpec(memory_space=pl.ANY),
            pl.BlockSpec(memory_space=pl.ANY),
        ],
        out_specs=pl.BlockSpec(memory_space=pltpu.SMEM),
        out_shape=jax.ShapeDtypeStruct((1, 1), jnp.float32),
        scratch_shapes=[
            pltpu.VMEM((_NBUF, _BR, _D), jnp.float32),
            pltpu.VMEM((_NBUF, _BR, _D), jnp.float32),
            pltpu.VMEM((_NBUF, _WR, _D), jnp.float32),
            pltpu.VMEM((_NBUF, _WR, _H), jnp.float32),
            pltpu.SMEM((3,), jnp.float32),
            pltpu.SemaphoreType.DMA((_NBUF,)),
            pltpu.SemaphoreType.DMA((_NBUF,)),
            pltpu.SemaphoreType.DMA((_NBUF,)),
            pltpu.SemaphoreType.DMA((_NBUF,)),
        ],
    )(y_hat, y, W1, W2)
    return out[0, 0]
